# Initial kernel scaffold; baseline (speedup 1.0000x reference)
import jax
import jax.numpy as jnp
from jax.experimental import pallas as pl


def kernel(x, W1, b1, g1, be1, W2, b2, g2, be2, in_idx, out_idx):
    raise NotImplementedError("write your pallas kernel here")


# speedup vs baseline: 2.6461x; 2.6461x over previous
import functools

import jax
import jax.numpy as jnp
from jax import lax
from jax.experimental import pallas as pl
from jax.experimental.pallas import tpu as pltpu
from jax.experimental.pallas import tpu_sc as plsc

_S = 128
_CH = 16384


def _make_gather(n_src, tot, c):
    nw = 32
    per_w = tot // nw
    nb = 5
    n_it = per_w // _S
    ngrp = n_it // nb
    assert per_w % _S == 0 and n_it % nb == 0
    mesh = plsc.VectorSubcoreMesh(core_axis_name="c", subcore_axis_name="s")

    @functools.partial(
        pl.kernel,
        out_type=jax.ShapeDtypeStruct((tot, c), jnp.float32),
        mesh=mesh,
        scratch_types=(
            [pltpu.VMEM((per_w,), jnp.int32)]
            + [pltpu.VMEM((_S, c), jnp.float32) for _ in range(nb)]
            + [pltpu.SemaphoreType.DMA, pltpu.SemaphoreType.DMA]
        ),
        compiler_params=pltpu.CompilerParams(use_tc_tiling_on_sc=False),
    )
    def gather_k(src, idxf, out, idx_v, *rest):
        bufs = rest[:nb]
        gsem, wsem = rest[nb], rest[nb + 1]
        cid = lax.axis_index("c")
        sid = lax.axis_index("s")
        wid = sid * 2 + cid
        base = pl.multiple_of(wid * per_w, _S)
        pltpu.sync_copy(idxf.at[pl.ds(base, per_w)], idx_v)

        def grp(g, carry):
            o = pl.multiple_of(g * (nb * _S), _S)
            ds = [
                pltpu.async_copy(
                    src.at[idx_v.at[pl.ds(o + b * _S, _S)]], bufs[b], gsem
                )
                for b in range(nb)
            ]
            for d in ds:
                d.wait()
            ws = [
                pltpu.async_copy(
                    bufs[b], out.at[pl.ds(base + o + b * _S, _S)], wsem
                )
                for b in range(nb)
            ]
            for w in ws:
                w.wait()
            return carry

        lax.fori_loop(0, ngrp, grp, 0, unroll=False)

    return gather_k


def _make_scatter(tot, pp, c, nch, n):
    mesh = plsc.VectorSubcoreMesh(core_axis_name="c", subcore_axis_name="s")
    rows_per_tile = _CH // 16

    @functools.partial(
        pl.kernel,
        out_type=jax.ShapeDtypeStruct((nch * _CH, c), jnp.float32),
        mesh=mesh,
        scratch_types=(
            pltpu.VMEM_SHARED((_CH + 8, c), jnp.float32),
            pltpu.VMEM((_S, c), jnp.float32),
            pltpu.VMEM((_S,), jnp.int32),
            pltpu.VMEM((_S,), jnp.int32),
            pltpu.VMEM((_S, c), jnp.float32),
            pltpu.VMEM((26 * (nch + 1) + 16,), jnp.int32),
        ),
        compiler_params=pltpu.CompilerParams(use_tc_tiling_on_sc=False),
    )
    def scat_k(msgs, oflat, dense, bnd, out, spm, stage, oid_v, lid_v, rows_v,
               bnd_v):
        cid = lax.axis_index("c")
        sid = lax.axis_index("s")
        pltpu.sync_copy(bnd, bnd_v.at[pl.ds(0, 26 * (nch + 1))])
        for ch in range(nch):
            @pl.when(cid == (ch % 2))
            def _chunk():
                def init_u(u, carry):
                    r0 = ch * _CH + sid * rows_per_tile + u * _S
                    l0 = sid * rows_per_tile + u * _S
                    pltpu.sync_copy(dense.at[pl.ds(r0, _S)], stage)
                    pltpu.sync_copy(stage, spm.at[pl.ds(l0, _S)])
                    return carry

                lax.fori_loop(0, rows_per_tile // _S, init_u, 0, unroll=False)
                plsc.subcore_barrier()

                def off_ki(ki, carry):
                    k = sid + 16 * ki

                    @pl.when(k < 26)
                    def _offset():
                        bv = bnd_v[pl.ds(k * (nch + 1), 16)]
                        lo = bv[ch]
                        hi = bv[ch + 1]
                        s0 = (lo // 8) * 8
                        nit = (hi - s0 + _S - 1) // _S

                        def it(i, c2):
                            s = s0 + i * _S
                            row0 = k * pp + s
                            pltpu.sync_copy(oflat.at[pl.ds(row0, _S)], oid_v)
                            pltpu.sync_copy(msgs.at[pl.ds(row0, _S)], rows_v)
                            for v in range(_S // 16):
                                vec = oid_v[pl.ds(v * 16, 16)]
                                pos = s + v * 16 + lax.broadcasted_iota(
                                    jnp.int32, (16,), 0)
                                ok = (pos >= lo) & (pos < hi)
                                lid_v[pl.ds(v * 16, 16)] = jnp.where(
                                    ok, vec - ch * _CH, _CH)
                            pltpu.sync_copy(rows_v, spm.at[lid_v], add=True)
                            return c2

                        lax.fori_loop(0, nit, it, 0, unroll=False)
                    return carry

                lax.fori_loop(0, 2, off_ki, 0, unroll=False)
                plsc.subcore_barrier()

                def wout_u(u, carry):
                    r0 = ch * _CH + sid * rows_per_tile + u * _S
                    l0 = sid * rows_per_tile + u * _S
                    pltpu.sync_copy(spm.at[pl.ds(l0, _S)], stage)
                    pltpu.sync_copy(stage, out.at[pl.ds(r0, _S)])
                    return carry

                lax.fori_loop(0, rows_per_tile // _S, wout_u, 0, unroll=False)

    return scat_k


def _msg_gemm(g2, w2, aff):
    tot2, _ = g2.shape
    noff = w2.shape[0]
    blk = 512
    jblk = tot2 // noff // blk

    def body(*refs):
        if aff is None:
            g_ref, w_ref, o_ref = refs
            g = g_ref[...]
        else:
            g_ref, w_ref, a_ref, o_ref = refs
            g = g_ref[...]
            g = jnp.maximum(g * a_ref[0:1, :] + a_ref[1:2, :], 0.0)
        o_ref[...] = jnp.dot(g, w_ref[0], preferred_element_type=jnp.float32)

    in_specs = [
        pl.BlockSpec((blk, 128), lambda k, j: (k * jblk + j, 0)),
        pl.BlockSpec((1, 128, 128), lambda k, j: (k, 0, 0)),
    ]
    args = [g2, w2]
    if aff is not None:
        in_specs.append(pl.BlockSpec((2, 128), lambda k, j: (0, 0)))
        args.append(aff)
    return pl.pallas_call(
        body,
        grid=(noff, jblk),
        in_specs=in_specs,
        out_specs=pl.BlockSpec((blk, 128), lambda k, j: (k * jblk + j, 0)),
        out_shape=jax.ShapeDtypeStruct(g2.shape, jnp.float32),
    )(*args)


def _dense_gemm(x2, w13, b128, aff, out_rows):
    blk = 512
    grid = (x2.shape[0] + blk - 1) // blk

    def body(*refs):
        if aff is None:
            x_ref, w_ref, b_ref, o_ref = refs
            v = x_ref[...]
        else:
            x_ref, w_ref, b_ref, a_ref, o_ref = refs
            v = x_ref[...]
            v = jnp.maximum(v * a_ref[0:1, :] + a_ref[1:2, :], 0.0)
        o_ref[...] = (
            jnp.dot(v, w_ref[...], preferred_element_type=jnp.float32)
            + b_ref[0:1, :]
        )

    in_specs = [
        pl.BlockSpec((blk, 128), lambda i: (i, 0)),
        pl.BlockSpec((128, 128), lambda i: (0, 0)),
        pl.BlockSpec((1, 128), lambda i: (0, 0)),
    ]
    args = [x2, w13, b128]
    if aff is not None:
        in_specs.append(pl.BlockSpec((2, 128), lambda i: (0, 0)))
        args.append(aff)
    return pl.pallas_call(
        body,
        grid=(grid,),
        in_specs=in_specs,
        out_specs=pl.BlockSpec((blk, 128), lambda i: (i, 0)),
        out_shape=jax.ShapeDtypeStruct((out_rows, 128), jnp.float32),
    )(*args)


def _bn_affine(hv, gvec, bevec, npair, nrows):
    blk = 512
    grid = hv.shape[0] // blk

    def body(h_ref, g_ref, be_ref, o_ref):
        i = pl.program_id(0)

        @pl.when(i == 0)
        def _init():
            o_ref[...] = jnp.zeros_like(o_ref)

        h = h_ref[...]
        r = i * blk + lax.broadcasted_iota(jnp.int32, (blk, 1), 0)
        h = jnp.where(r < npair, h, 0.0)
        acc = jnp.concatenate(
            [jnp.sum(h, 0, keepdims=True), jnp.sum(h * h, 0, keepdims=True)], 0
        )
        o_ref[...] += acc

        @pl.when(i == grid - 1)
        def _fin():
            s = o_ref[0:1, :]
            q = o_ref[1:2, :]
            s64 = s[:, :64] + s[:, 64:]
            q64 = q[:, :64] + q[:, 64:]
            m = s64 / nrows
            var = q64 / nrows - m * m
            inv = lax.rsqrt(var + 1e-5)
            sc = inv * g_ref[...]
            sh = be_ref[...] - m * sc
            o_ref[...] = jnp.concatenate(
                [jnp.concatenate([sc, sc], 1), jnp.concatenate([sh, sh], 1)], 0
            )

    return pl.pallas_call(
        body,
        grid=(grid,),
        in_specs=[
            pl.BlockSpec((blk, 128), lambda i: (i, 0)),
            pl.BlockSpec((1, 64), lambda i: (0, 0)),
            pl.BlockSpec((1, 64), lambda i: (0, 0)),
        ],
        out_specs=pl.BlockSpec((2, 128), lambda i: (0, 0)),
        out_shape=jax.ShapeDtypeStruct((2, 128), jnp.float32),
    )(hv, gvec.reshape(1, 64), bevec.reshape(1, 64))


def _final(h2v, x2, aff2, npair):
    blk = 512
    grid = (npair + blk - 1) // blk

    def body(h_ref, x_ref, a_ref, o_ref):
        h = h_ref[...]
        o_ref[...] = jnp.maximum(
            h * a_ref[0:1, :] + a_ref[1:2, :] + x_ref[...], 0.0
        )

    return pl.pallas_call(
        body,
        grid=(grid,),
        in_specs=[
            pl.BlockSpec((blk, 128), lambda i: (i, 0)),
            pl.BlockSpec((blk, 128), lambda i: (i, 0)),
            pl.BlockSpec((2, 128), lambda i: (0, 0)),
        ],
        out_specs=pl.BlockSpec((blk, 128), lambda i: (i, 0)),
        out_shape=jax.ShapeDtypeStruct((npair, 128), jnp.float32),
    )(h2v, x2, aff2)


def kernel(x, W1, b1, g1, be1, W2, b2, g2, be2, in_idx, out_idx):
    n, c = x.shape
    p = in_idx.shape[1]
    pp = -(-(p + _S) // 1024) * 1024
    tot = 26 * pp
    nch = -(-(n + 1) // _CH)
    hp = nch * _CH
    npair = n // 2

    in_pad = jnp.full((26, pp), n - 1, jnp.int32)
    in_pad = in_pad.at[:, :p].set(jnp.minimum(in_idx, n - 1))
    in_flat = in_pad.reshape(-1)
    out_pad = jnp.full((26, pp), n, jnp.int32)
    out_pad = out_pad.at[:, :p].set(out_idx)
    out_flat = out_pad.reshape(-1)
    edges = jnp.array([min(i * _CH, n) for i in range(nch + 1)], jnp.int32)
    bounds = jnp.sum(
        out_pad[:, :, None] < edges[None, None, :], axis=1, dtype=jnp.int32
    ).reshape(-1)

    sel = [k for k in range(27) if k != 13]

    def blockdiag(w):
        z = jnp.zeros(w.shape[:-2] + (64, 64), w.dtype)
        top = jnp.concatenate([w, z], -1)
        bot = jnp.concatenate([z, w], -1)
        return jnp.concatenate([top, bot], -2)

    w1n = blockdiag(W1[jnp.array(sel)])
    w2n = blockdiag(W2[jnp.array(sel)])
    w1c = blockdiag(W1[13])
    w2c = blockdiag(W2[13])
    b1d = jnp.tile(b1, 2).reshape(1, 128)
    b2d = jnp.tile(b2, 2).reshape(1, 128)

    x2 = x.reshape(npair, 128)

    gather_x = _make_gather(n, tot, c)
    gather_h = _make_gather(hp, tot, c)
    scatter = _make_scatter(tot, pp, c, nch, n)

    g_rows = gather_x(x, in_flat)
    m1 = _msg_gemm(g_rows.reshape(tot // 2, 128), w1n, None)
    d1 = _dense_gemm(x2, w1c, b1d, None, hp // 2)
    h1 = scatter(m1.reshape(tot, c), out_flat, d1.reshape(hp, c), bounds)
    aff1 = _bn_affine(h1.reshape(hp // 2, 128), g1, be1, npair, n)

    g_rows2 = gather_h(h1, in_flat)
    m2 = _msg_gemm(g_rows2.reshape(tot // 2, 128), w2n, aff1)
    d2 = _dense_gemm(h1.reshape(hp // 2, 128), w2c, b2d, aff1, hp // 2)
    h2 = scatter(m2.reshape(tot, c), out_flat, d2.reshape(hp, c), bounds)
    aff2 = _bn_affine(h2.reshape(hp // 2, 128), g2, be2, npair, n)

    out = _final(h2.reshape(hp // 2, 128), x2, aff2, npair)
    return out.reshape(n, c)


# speedup vs baseline: 2.8558x; 1.0793x over previous
import functools

import jax
import jax.numpy as jnp
from jax import lax
from jax.experimental import pallas as pl
from jax.experimental.pallas import tpu as pltpu
from jax.experimental.pallas import tpu_sc as plsc

_S = 128
_CH = 8192


def _make_gather(n_src, tot, c):
    nw = 32
    per_w = tot // nw
    gpb = 5
    gw = gpb * _S
    ngrp = per_w // gw
    assert per_w % gw == 0
    mesh = plsc.VectorSubcoreMesh(core_axis_name="c", subcore_axis_name="s")

    @functools.partial(
        pl.kernel,
        out_type=jax.ShapeDtypeStruct((tot, c), jnp.float32),
        mesh=mesh,
        scratch_types=(
            pltpu.VMEM((per_w,), jnp.int32),
            pltpu.VMEM((gw, c), jnp.float32),
            pltpu.VMEM((gw, c), jnp.float32),
            pltpu.SemaphoreType.DMA,
            pltpu.SemaphoreType.DMA,
        ),
        compiler_params=pltpu.CompilerParams(use_tc_tiling_on_sc=False),
    )
    def gather_k(src, idxf, out, idx_v, buf0, buf1, gsem, wsem):
        cid = lax.axis_index("c")
        sid = lax.axis_index("s")
        wid = sid * 2 + cid
        base = pl.multiple_of(wid * per_w, _S)
        pltpu.sync_copy(idxf.at[pl.ds(base, per_w)], idx_v)

        def issue(g, buf):
            return [
                pltpu.async_copy(
                    src.at[idx_v.at[pl.ds(g * gw + b * _S, _S)]],
                    buf.at[pl.ds(b * _S, _S)],
                    gsem,
                )
                for b in range(gpb)
            ]

        descs = issue(0, buf0)
        wprev = None
        for g in range(ngrp):
            cur, nxt = (buf0, buf1) if g % 2 == 0 else (buf1, buf0)
            for d in descs:
                d.wait()
            if g + 1 < ngrp:
                descs = issue(g + 1, nxt)
            if wprev is not None:
                wprev.wait()
            wprev = pltpu.async_copy(
                cur, out.at[pl.ds(base + g * gw, gw)], wsem
            )
        wprev.wait()

    return gather_k


def _make_scatter(tot, pp, c, nch, n):
    mesh = plsc.VectorSubcoreMesh(core_axis_name="c", subcore_axis_name="s")
    rows_per_tile = _CH // 16

    @functools.partial(
        pl.kernel,
        out_type=jax.ShapeDtypeStruct((nch * _CH, c), jnp.float32),
        mesh=mesh,
        scratch_types=(
            pltpu.VMEM_SHARED((_CH + 8, c), jnp.float32),
            pltpu.VMEM((rows_per_tile, c), jnp.float32),
            pltpu.VMEM((_S,), jnp.int32),
            pltpu.VMEM((_S,), jnp.int32),
            pltpu.VMEM((_S, c), jnp.float32),
            pltpu.VMEM((448,), jnp.int32),
            pltpu.VMEM((448,), jnp.int32),
            pltpu.SemaphoreType.DMA,
        ),
        compiler_params=pltpu.CompilerParams(use_tc_tiling_on_sc=False),
    )
    def scat_k(msgs, oflat, dense, bnd, ps, out, spm, stage, oid_v, lid_v,
               rows_v, bnd_v, ps_v, lsem):
        cid = lax.axis_index("c")
        sid = lax.axis_index("s")
        pltpu.sync_copy(bnd, bnd_v)
        pltpu.sync_copy(ps, ps_v)
        for ch in range(nch):
            @pl.when(cid == (ch % 2))
            def _chunk():
                r0 = ch * _CH + sid * rows_per_tile
                l0 = sid * rows_per_tile
                pltpu.sync_copy(dense.at[pl.ds(r0, rows_per_tile)], stage)
                pltpu.sync_copy(stage, spm.at[pl.ds(l0, rows_per_tile)])
                plsc.subcore_barrier()

                tvec = ps_v[pl.ds(26 * 16, 16)]
                total = tvec[ch]
                t0 = total * sid // 16
                t1 = total * (sid + 1) // 16

                def k_body(k, carry):
                    bv = bnd_v[pl.ds(k * 16, 16)]
                    pk = ps_v[pl.ds(k * 16, 16)]
                    pk1 = ps_v[pl.ds(k * 16 + 16, 16)]
                    lo = bv[ch]
                    p0 = pk[ch]
                    p1 = pk1[ch]
                    o_lo = jnp.maximum(p0, t0)
                    o_hi = jnp.minimum(p1, t1)

                    @pl.when(o_lo < o_hi)
                    def _seg():
                        s_lo = lo + (o_lo - p0)
                        s_hi = lo + (o_hi - p0)
                        s0 = (s_lo // 8) * 8
                        nit = (s_hi - s0 + _S - 1) // _S

                        def it(i, c2):
                            s = s0 + i * _S
                            row0 = k * pp + s
                            d1 = pltpu.async_copy(
                                oflat.at[pl.ds(row0, _S)], oid_v, lsem)
                            d2 = pltpu.async_copy(
                                msgs.at[pl.ds(row0, _S)], rows_v, lsem)
                            d1.wait()
                            d2.wait()
                            for v in range(_S // 16):
                                vec = oid_v[pl.ds(v * 16, 16)]
                                pos = s + v * 16 + lax.broadcasted_iota(
                                    jnp.int32, (16,), 0)
                                ok = (pos >= s_lo) & (pos < s_hi)
                                lid_v[pl.ds(v * 16, 16)] = jnp.where(
                                    ok, vec - ch * _CH, _CH)
                            pltpu.sync_copy(rows_v, spm.at[lid_v], add=True)
                            return c2

                        lax.fori_loop(0, nit, it, 0, unroll=False)
                    return carry

                lax.fori_loop(0, 26, k_body, 0, unroll=False)
                plsc.subcore_barrier()
                pltpu.sync_copy(spm.at[pl.ds(l0, rows_per_tile)], stage)
                pltpu.sync_copy(stage, out.at[pl.ds(r0, rows_per_tile)])

    return scat_k


def _msg_gemm(g2, w2, aff):
    tot2, _ = g2.shape
    noff = w2.shape[0]
    blk = 512
    jblk = tot2 // noff // blk

    def body(*refs):
        if aff is None:
            g_ref, w_ref, o_ref = refs
            g = g_ref[...]
        else:
            g_ref, w_ref, a_ref, o_ref = refs
            g = g_ref[...]
            g = jnp.maximum(g * a_ref[0:1, :] + a_ref[1:2, :], 0.0)
        o_ref[...] = jnp.dot(g, w_ref[0], preferred_element_type=jnp.float32)

    in_specs = [
        pl.BlockSpec((blk, 128), lambda k, j: (k * jblk + j, 0)),
        pl.BlockSpec((1, 128, 128), lambda k, j: (k, 0, 0)),
    ]
    args = [g2, w2]
    if aff is not None:
        in_specs.append(pl.BlockSpec((2, 128), lambda k, j: (0, 0)))
        args.append(aff)
    return pl.pallas_call(
        body,
        grid=(noff, jblk),
        in_specs=in_specs,
        out_specs=pl.BlockSpec((blk, 128), lambda k, j: (k * jblk + j, 0)),
        out_shape=jax.ShapeDtypeStruct(g2.shape, jnp.float32),
    )(*args)


def _dense_gemm(x2, w13, b128, aff, out_rows):
    blk = 512
    grid = (x2.shape[0] + blk - 1) // blk

    def body(*refs):
        if aff is None:
            x_ref, w_ref, b_ref, o_ref = refs
            v = x_ref[...]
        else:
            x_ref, w_ref, b_ref, a_ref, o_ref = refs
            v = x_ref[...]
            v = jnp.maximum(v * a_ref[0:1, :] + a_ref[1:2, :], 0.0)
        o_ref[...] = (
            jnp.dot(v, w_ref[...], preferred_element_type=jnp.float32)
            + b_ref[0:1, :]
        )

    in_specs = [
        pl.BlockSpec((blk, 128), lambda i: (i, 0)),
        pl.BlockSpec((128, 128), lambda i: (0, 0)),
        pl.BlockSpec((1, 128), lambda i: (0, 0)),
    ]
    args = [x2, w13, b128]
    if aff is not None:
        in_specs.append(pl.BlockSpec((2, 128), lambda i: (0, 0)))
        args.append(aff)
    return pl.pallas_call(
        body,
        grid=(grid,),
        in_specs=in_specs,
        out_specs=pl.BlockSpec((blk, 128), lambda i: (i, 0)),
        out_shape=jax.ShapeDtypeStruct((out_rows, 128), jnp.float32),
    )(*args)


def _bn_affine(hv, gvec, bevec, npair, nrows):
    blk = 512
    grid = hv.shape[0] // blk

    def body(h_ref, g_ref, be_ref, o_ref):
        i = pl.program_id(0)

        @pl.when(i == 0)
        def _init():
            o_ref[...] = jnp.zeros_like(o_ref)

        h = h_ref[...]
        r = i * blk + lax.broadcasted_iota(jnp.int32, (blk, 1), 0)
        h = jnp.where(r < npair, h, 0.0)
        acc = jnp.concatenate(
            [jnp.sum(h, 0, keepdims=True), jnp.sum(h * h, 0, keepdims=True)], 0
        )
        o_ref[...] += acc

        @pl.when(i == grid - 1)
        def _fin():
            s = o_ref[0:1, :]
            q = o_ref[1:2, :]
            s64 = s[:, :64] + s[:, 64:]
            q64 = q[:, :64] + q[:, 64:]
            m = s64 / nrows
            var = q64 / nrows - m * m
            inv = lax.rsqrt(var + 1e-5)
            sc = inv * g_ref[...]
            sh = be_ref[...] - m * sc
            o_ref[...] = jnp.concatenate(
                [jnp.concatenate([sc, sc], 1), jnp.concatenate([sh, sh], 1)], 0
            )

    return pl.pallas_call(
        body,
        grid=(grid,),
        in_specs=[
            pl.BlockSpec((blk, 128), lambda i: (i, 0)),
            pl.BlockSpec((1, 64), lambda i: (0, 0)),
            pl.BlockSpec((1, 64), lambda i: (0, 0)),
        ],
        out_specs=pl.BlockSpec((2, 128), lambda i: (0, 0)),
        out_shape=jax.ShapeDtypeStruct((2, 128), jnp.float32),
    )(hv, gvec.reshape(1, 64), bevec.reshape(1, 64))


def _final(h2v, x2, aff2, npair):
    blk = 512
    grid = (npair + blk - 1) // blk

    def body(h_ref, x_ref, a_ref, o_ref):
        h = h_ref[...]
        o_ref[...] = jnp.maximum(
            h * a_ref[0:1, :] + a_ref[1:2, :] + x_ref[...], 0.0
        )

    return pl.pallas_call(
        body,
        grid=(grid,),
        in_specs=[
            pl.BlockSpec((blk, 128), lambda i: (i, 0)),
            pl.BlockSpec((blk, 128), lambda i: (i, 0)),
            pl.BlockSpec((2, 128), lambda i: (0, 0)),
        ],
        out_specs=pl.BlockSpec((blk, 128), lambda i: (i, 0)),
        out_shape=jax.ShapeDtypeStruct((npair, 128), jnp.float32),
    )(h2v, x2, aff2)


def kernel(x, W1, b1, g1, be1, W2, b2, g2, be2, in_idx, out_idx):
    n, c = x.shape
    p = in_idx.shape[1]
    pp = -(-(p + _S) // 1024) * 1024
    tot = 26 * pp
    nch = -(-(n + 1) // _CH)
    hp = nch * _CH
    npair = n // 2

    in_pad = jnp.full((26, pp), n - 1, jnp.int32)
    in_pad = in_pad.at[:, :p].set(jnp.minimum(in_idx, n - 1))
    in_flat = in_pad.reshape(-1)
    out_pad = jnp.full((26, pp), n, jnp.int32)
    out_pad = out_pad.at[:, :p].set(out_idx)
    out_flat = out_pad.reshape(-1)
    edges = jnp.array([min(i * _CH, n) for i in range(nch + 1)], jnp.int32)
    bounds2d = jnp.sum(
        out_pad[:, :, None] < edges[None, None, :], axis=1, dtype=jnp.int32
    )
    bnd_flat = (
        jnp.zeros((28, 16), jnp.int32).at[:26, : nch + 1].set(bounds2d).reshape(-1)
    )
    lens = bounds2d[:, 1:] - bounds2d[:, :-1]
    cs = jnp.concatenate(
        [jnp.zeros((1, nch), jnp.int32), jnp.cumsum(lens, 0, dtype=jnp.int32)], 0
    )
    ps_flat = (
        jnp.zeros((28, 16), jnp.int32).at[:27, :nch].set(cs).reshape(-1)
    )

    sel = [k for k in range(27) if k != 13]

    def blockdiag(w):
        z = jnp.zeros(w.shape[:-2] + (64, 64), w.dtype)
        top = jnp.concatenate([w, z], -1)
        bot = jnp.concatenate([z, w], -1)
        return jnp.concatenate([top, bot], -2)

    w1n = blockdiag(W1[jnp.array(sel)])
    w2n = blockdiag(W2[jnp.array(sel)])
    w1c = blockdiag(W1[13])
    w2c = blockdiag(W2[13])
    b1d = jnp.tile(b1, 2).reshape(1, 128)
    b2d = jnp.tile(b2, 2).reshape(1, 128)

    x2 = x.reshape(npair, 128)

    gather_x = _make_gather(n, tot, c)
    gather_h = _make_gather(hp, tot, c)
    scatter = _make_scatter(tot, pp, c, nch, n)

    g_rows = gather_x(x, in_flat)
    m1 = _msg_gemm(g_rows.reshape(tot // 2, 128), w1n, None)
    d1 = _dense_gemm(x2, w1c, b1d, None, hp // 2)
    h1 = scatter(m1.reshape(tot, c), out_flat, d1.reshape(hp, c), bnd_flat,
                 ps_flat)
    aff1 = _bn_affine(h1.reshape(hp // 2, 128), g1, be1, npair, n)

    g_rows2 = gather_h(h1, in_flat)
    m2 = _msg_gemm(g_rows2.reshape(tot // 2, 128), w2n, aff1)
    d2 = _dense_gemm(h1.reshape(hp // 2, 128), w2c, b2d, aff1, hp // 2)
    h2 = scatter(m2.reshape(tot, c), out_flat, d2.reshape(hp, c), bnd_flat,
                 ps_flat)
    aff2 = _bn_affine(h2.reshape(hp // 2, 128), g2, be2, npair, n)

    out = _final(h2.reshape(hp // 2, 128), x2, aff2, npair)
    return out.reshape(n, c)


# speedup vs baseline: 4.0827x; 1.4296x over previous
import functools

import jax
import jax.numpy as jnp
from jax import lax
from jax.experimental import pallas as pl
from jax.experimental.pallas import tpu as pltpu
from jax.experimental.pallas import tpu_sc as plsc

_S = 128
_CH = 8192


def _make_gather(n_src, tot, c, s=_S, gpb=5):
    _S = s
    nw = 32
    per_w = tot // nw
    gw = gpb * _S
    ngrp = per_w // gw
    assert per_w % gw == 0
    mesh = plsc.VectorSubcoreMesh(core_axis_name="c", subcore_axis_name="s")

    @functools.partial(
        pl.kernel,
        out_type=jax.ShapeDtypeStruct((tot, c), jnp.float32),
        mesh=mesh,
        scratch_types=(
            pltpu.VMEM((per_w,), jnp.int32),
            pltpu.VMEM((gw, c), jnp.float32),
            pltpu.VMEM((gw, c), jnp.float32),
            pltpu.SemaphoreType.DMA,
            pltpu.SemaphoreType.DMA,
        ),
        compiler_params=pltpu.CompilerParams(use_tc_tiling_on_sc=False),
    )
    def gather_k(src, idxf, out, idx_v, buf0, buf1, gsem, wsem):
        cid = lax.axis_index("c")
        sid = lax.axis_index("s")
        wid = sid * 2 + cid
        base = pl.multiple_of(wid * per_w, _S)
        pltpu.sync_copy(idxf.at[pl.ds(base, per_w)], idx_v)

        def issue(g, buf):
            return [
                pltpu.async_copy(
                    src.at[idx_v.at[pl.ds(g * gw + b * _S, _S)]],
                    buf.at[pl.ds(b * _S, _S)],
                    gsem,
                )
                for b in range(gpb)
            ]

        descs = issue(0, buf0)
        wprev = None
        for g in range(ngrp):
            cur, nxt = (buf0, buf1) if g % 2 == 0 else (buf1, buf0)
            for d in descs:
                d.wait()
            if g + 1 < ngrp:
                descs = issue(g + 1, nxt)
            if wprev is not None:
                wprev.wait()
            wprev = pltpu.async_copy(
                cur, out.at[pl.ds(base + g * gw, gw)], wsem
            )
        wprev.wait()

    return gather_k


def _make_scatter(tot, pp, c, nch, n):
    mesh = plsc.VectorSubcoreMesh(core_axis_name="c", subcore_axis_name="s")
    rows_per_tile = _CH // 16

    @functools.partial(
        pl.kernel,
        out_type=jax.ShapeDtypeStruct((nch * _CH, c), jnp.float32),
        mesh=mesh,
        scratch_types=(
            pltpu.VMEM_SHARED((_CH + 8, c), jnp.float32),
            pltpu.VMEM((rows_per_tile, c), jnp.float32),
            pltpu.VMEM((_S,), jnp.int32),
            pltpu.VMEM((_S,), jnp.int32),
            pltpu.VMEM((_S, c), jnp.float32),
            pltpu.VMEM((448,), jnp.int32),
            pltpu.VMEM((448,), jnp.int32),
            pltpu.SemaphoreType.DMA,
        ),
        compiler_params=pltpu.CompilerParams(use_tc_tiling_on_sc=False),
    )
    def scat_k(msgs, oflat, dense, bnd, ps, out, spm, stage, oid_v, lid_v,
               rows_v, bnd_v, ps_v, lsem):
        cid = lax.axis_index("c")
        sid = lax.axis_index("s")
        pltpu.sync_copy(bnd, bnd_v)
        pltpu.sync_copy(ps, ps_v)
        for ch in range(nch):
            @pl.when(cid == (ch % 2))
            def _chunk():
                r0 = ch * _CH + sid * rows_per_tile
                l0 = sid * rows_per_tile
                pltpu.sync_copy(dense.at[pl.ds(r0, rows_per_tile)], stage)
                pltpu.sync_copy(stage, spm.at[pl.ds(l0, rows_per_tile)])
                plsc.subcore_barrier()

                tvec = ps_v[pl.ds(26 * 16, 16)]
                total = tvec[ch]
                t0 = total * sid // 16
                t1 = total * (sid + 1) // 16

                def k_body(k, carry):
                    bv = bnd_v[pl.ds(k * 16, 16)]
                    pk = ps_v[pl.ds(k * 16, 16)]
                    pk1 = ps_v[pl.ds(k * 16 + 16, 16)]
                    lo = bv[ch]
                    p0 = pk[ch]
                    p1 = pk1[ch]
                    o_lo = jnp.maximum(p0, t0)
                    o_hi = jnp.minimum(p1, t1)

                    @pl.when(o_lo < o_hi)
                    def _seg():
                        s_lo = lo + (o_lo - p0)
                        s_hi = lo + (o_hi - p0)
                        s0 = (s_lo // 8) * 8
                        nit = (s_hi - s0 + _S - 1) // _S

                        def it(i, c2):
                            s = s0 + i * _S
                            row0 = k * pp + s
                            d1 = pltpu.async_copy(
                                oflat.at[pl.ds(row0, _S)], oid_v, lsem)
                            d2 = pltpu.async_copy(
                                msgs.at[pl.ds(row0, _S)], rows_v, lsem)
                            d1.wait()
                            d2.wait()
                            for v in range(_S // 16):
                                vec = oid_v[pl.ds(v * 16, 16)]
                                pos = s + v * 16 + lax.broadcasted_iota(
                                    jnp.int32, (16,), 0)
                                ok = (pos >= s_lo) & (pos < s_hi)
                                lid_v[pl.ds(v * 16, 16)] = jnp.where(
                                    ok, vec - ch * _CH, _CH + (pos & 7))
                            pltpu.sync_copy(rows_v, spm.at[lid_v], add=True)
                            return c2

                        lax.fori_loop(0, nit, it, 0, unroll=False)
                    return carry

                lax.fori_loop(0, 26, k_body, 0, unroll=False)
                plsc.subcore_barrier()
                pltpu.sync_copy(spm.at[pl.ds(l0, rows_per_tile)], stage)
                pltpu.sync_copy(stage, out.at[pl.ds(r0, rows_per_tile)])

    return scat_k


def _msg_gemm(g2, w2, aff):
    tot2, _ = g2.shape
    noff = w2.shape[0]
    blk = 512
    jblk = tot2 // noff // blk

    def body(*refs):
        if aff is None:
            g_ref, w_ref, o_ref = refs
            g = g_ref[...]
        else:
            g_ref, w_ref, a_ref, o_ref = refs
            g = g_ref[...]
            g = jnp.maximum(g * a_ref[0:1, :] + a_ref[1:2, :], 0.0)
        o_ref[...] = jnp.dot(g, w_ref[0], preferred_element_type=jnp.float32)

    in_specs = [
        pl.BlockSpec((blk, 128), lambda k, j: (k * jblk + j, 0)),
        pl.BlockSpec((1, 128, 128), lambda k, j: (k, 0, 0)),
    ]
    args = [g2, w2]
    if aff is not None:
        in_specs.append(pl.BlockSpec((2, 128), lambda k, j: (0, 0)))
        args.append(aff)
    return pl.pallas_call(
        body,
        grid=(noff, jblk),
        in_specs=in_specs,
        out_specs=pl.BlockSpec((blk, 128), lambda k, j: (k * jblk + j, 0)),
        out_shape=jax.ShapeDtypeStruct(g2.shape, jnp.float32),
    )(*args)


def _dense_gemm(x2, w13, b128, aff, out_rows):
    blk = 512
    grid = (x2.shape[0] + blk - 1) // blk

    def body(*refs):
        if aff is None:
            x_ref, w_ref, b_ref, o_ref = refs
            v = x_ref[...]
        else:
            x_ref, w_ref, b_ref, a_ref, o_ref = refs
            v = x_ref[...]
            v = jnp.maximum(v * a_ref[0:1, :] + a_ref[1:2, :], 0.0)
        o_ref[...] = (
            jnp.dot(v, w_ref[...], preferred_element_type=jnp.float32)
            + b_ref[0:1, :]
        )

    in_specs = [
        pl.BlockSpec((blk, 128), lambda i: (i, 0)),
        pl.BlockSpec((128, 128), lambda i: (0, 0)),
        pl.BlockSpec((1, 128), lambda i: (0, 0)),
    ]
    args = [x2, w13, b128]
    if aff is not None:
        in_specs.append(pl.BlockSpec((2, 128), lambda i: (0, 0)))
        args.append(aff)
    return pl.pallas_call(
        body,
        grid=(grid,),
        in_specs=in_specs,
        out_specs=pl.BlockSpec((blk, 128), lambda i: (i, 0)),
        out_shape=jax.ShapeDtypeStruct((out_rows, 128), jnp.float32),
    )(*args)


def _bn_affine(hv, gvec, bevec, npair, nrows):
    blk = 512
    grid = hv.shape[0] // blk

    def body(h_ref, g_ref, be_ref, o_ref):
        i = pl.program_id(0)

        @pl.when(i == 0)
        def _init():
            o_ref[...] = jnp.zeros_like(o_ref)

        h = h_ref[...]
        r = i * blk + lax.broadcasted_iota(jnp.int32, (blk, 1), 0)
        h = jnp.where(r < npair, h, 0.0)
        acc = jnp.concatenate(
            [jnp.sum(h, 0, keepdims=True), jnp.sum(h * h, 0, keepdims=True)], 0
        )
        o_ref[...] += acc

        @pl.when(i == grid - 1)
        def _fin():
            s = o_ref[0:1, :]
            q = o_ref[1:2, :]
            s64 = s[:, :64] + s[:, 64:]
            q64 = q[:, :64] + q[:, 64:]
            m = s64 / nrows
            var = q64 / nrows - m * m
            inv = lax.rsqrt(var + 1e-5)
            sc = inv * g_ref[...]
            sh = be_ref[...] - m * sc
            o_ref[...] = jnp.concatenate(
                [jnp.concatenate([sc, sc], 1), jnp.concatenate([sh, sh], 1)], 0
            )

    return pl.pallas_call(
        body,
        grid=(grid,),
        in_specs=[
            pl.BlockSpec((blk, 128), lambda i: (i, 0)),
            pl.BlockSpec((1, 64), lambda i: (0, 0)),
            pl.BlockSpec((1, 64), lambda i: (0, 0)),
        ],
        out_specs=pl.BlockSpec((2, 128), lambda i: (0, 0)),
        out_shape=jax.ShapeDtypeStruct((2, 128), jnp.float32),
    )(hv, gvec.reshape(1, 64), bevec.reshape(1, 64))


def _final(h2v, x2, aff2, npair):
    blk = 512
    grid = (npair + blk - 1) // blk

    def body(h_ref, x_ref, a_ref, o_ref):
        h = h_ref[...]
        o_ref[...] = jnp.maximum(
            h * a_ref[0:1, :] + a_ref[1:2, :] + x_ref[...], 0.0
        )

    return pl.pallas_call(
        body,
        grid=(grid,),
        in_specs=[
            pl.BlockSpec((blk, 128), lambda i: (i, 0)),
            pl.BlockSpec((blk, 128), lambda i: (i, 0)),
            pl.BlockSpec((2, 128), lambda i: (0, 0)),
        ],
        out_specs=pl.BlockSpec((blk, 128), lambda i: (i, 0)),
        out_shape=jax.ShapeDtypeStruct((npair, 128), jnp.float32),
    )(h2v, x2, aff2)


def kernel(x, W1, b1, g1, be1, W2, b2, g2, be2, in_idx, out_idx):
    n, c = x.shape
    p = in_idx.shape[1]
    pp = -(-(p + _S) // 1024) * 1024
    tot = 26 * pp
    nch = -(-(n + 1) // _CH)
    hp = nch * _CH
    npair = n // 2

    in_pad = jnp.full((26, pp), n, jnp.int32)
    in_pad = in_pad.at[:, :p].set(in_idx)
    in_flat = in_pad.reshape(-1)
    spread = (jnp.arange(tot, dtype=jnp.int32) * 7919) % n
    in_flat = jnp.where(in_flat >= n, spread, in_flat)
    out_pad = jnp.full((26, pp), n, jnp.int32)
    out_pad = out_pad.at[:, :p].set(out_idx)
    out_flat = out_pad.reshape(-1)
    edges = jnp.array([min(i * _CH, n) for i in range(nch + 1)], jnp.int32)
    bounds2d = jnp.sum(
        out_pad[:, :, None] < edges[None, None, :], axis=1, dtype=jnp.int32
    )
    bnd_flat = (
        jnp.zeros((28, 16), jnp.int32).at[:26, : nch + 1].set(bounds2d).reshape(-1)
    )
    lens = bounds2d[:, 1:] - bounds2d[:, :-1]
    cs = jnp.concatenate(
        [jnp.zeros((1, nch), jnp.int32), jnp.cumsum(lens, 0, dtype=jnp.int32)], 0
    )
    ps_flat = (
        jnp.zeros((28, 16), jnp.int32).at[:27, :nch].set(cs).reshape(-1)
    )

    sel = [k for k in range(27) if k != 13]

    def blockdiag(w):
        z = jnp.zeros(w.shape[:-2] + (64, 64), w.dtype)
        top = jnp.concatenate([w, z], -1)
        bot = jnp.concatenate([z, w], -1)
        return jnp.concatenate([top, bot], -2)

    w1n = blockdiag(W1[jnp.array(sel)])
    w2n = blockdiag(W2[jnp.array(sel)])
    w1c = blockdiag(W1[13])
    w2c = blockdiag(W2[13])
    b1d = jnp.tile(b1, 2).reshape(1, 128)
    b2d = jnp.tile(b2, 2).reshape(1, 128)

    x2 = x.reshape(npair, 128)

    gather_x = _make_gather(n, tot, c)
    gather_h = _make_gather(hp, tot, c)
    scatter = _make_scatter(tot, pp, c, nch, n)

    g_rows = gather_x(x, in_flat)
    m1 = _msg_gemm(g_rows.reshape(tot // 2, 128), w1n, None)
    d1 = _dense_gemm(x2, w1c, b1d, None, hp // 2)
    h1 = scatter(m1.reshape(tot, c), out_flat, d1.reshape(hp, c), bnd_flat,
                 ps_flat)
    aff1 = _bn_affine(h1.reshape(hp // 2, 128), g1, be1, npair, n)

    g_rows2 = gather_h(h1, in_flat)
    m2 = _msg_gemm(g_rows2.reshape(tot // 2, 128), w2n, aff1)
    d2 = _dense_gemm(h1.reshape(hp // 2, 128), w2c, b2d, aff1, hp // 2)
    h2 = scatter(m2.reshape(tot, c), out_flat, d2.reshape(hp, c), bnd_flat,
                 ps_flat)
    aff2 = _bn_affine(h2.reshape(hp // 2, 128), g2, be2, npair, n)

    out = _final(h2.reshape(hp // 2, 128), x2, aff2, npair)
    return out.reshape(n, c)


# speedup vs baseline: 5.4809x; 1.3425x over previous
import functools

import jax
import jax.numpy as jnp
from jax import lax
from jax.experimental import pallas as pl
from jax.experimental.pallas import tpu as pltpu
from jax.experimental.pallas import tpu_sc as plsc

_S = 128
_CH = 8192


def _make_gather(n_src, tot, c, s=_S, gpb=5):
    _S = s
    nw = 32
    per_w = tot // nw
    gw = gpb * _S
    ngrp = per_w // gw
    assert per_w % gw == 0
    mesh = plsc.VectorSubcoreMesh(core_axis_name="c", subcore_axis_name="s")

    @functools.partial(
        pl.kernel,
        out_type=jax.ShapeDtypeStruct((tot, c), jnp.float32),
        mesh=mesh,
        scratch_types=(
            pltpu.VMEM((per_w,), jnp.int32),
            pltpu.VMEM((gw, c), jnp.float32),
            pltpu.VMEM((gw, c), jnp.float32),
            pltpu.SemaphoreType.DMA,
            pltpu.SemaphoreType.DMA,
        ),
        compiler_params=pltpu.CompilerParams(use_tc_tiling_on_sc=False),
    )
    def gather_k(src, idxf, out, idx_v, buf0, buf1, gsem, wsem):
        cid = lax.axis_index("c")
        sid = lax.axis_index("s")
        wid = sid * 2 + cid
        base = pl.multiple_of(wid * per_w, _S)
        pltpu.sync_copy(idxf.at[pl.ds(base, per_w)], idx_v)

        def issue(g, buf):
            return [
                pltpu.async_copy(
                    src.at[idx_v.at[pl.ds(g * gw + b * _S, _S)]],
                    buf.at[pl.ds(b * _S, _S)],
                    gsem,
                )
                for b in range(gpb)
            ]

        descs = issue(0, buf0)
        wprev = None
        for g in range(ngrp):
            cur, nxt = (buf0, buf1) if g % 2 == 0 else (buf1, buf0)
            for d in descs:
                d.wait()
            if g + 1 < ngrp:
                descs = issue(g + 1, nxt)
            if wprev is not None:
                wprev.wait()
            wprev = pltpu.async_copy(
                cur, out.at[pl.ds(base + g * gw, gw)], wsem
            )
        wprev.wait()

    return gather_k


def _make_scatter(tot, pp, c, nch, n):
    mesh = plsc.VectorSubcoreMesh(core_axis_name="c", subcore_axis_name="s")
    rows_per_tile = _CH // 16

    @functools.partial(
        pl.kernel,
        out_type=jax.ShapeDtypeStruct((nch * _CH, c), jnp.float32),
        mesh=mesh,
        scratch_types=(
            pltpu.VMEM_SHARED((_CH + 8, c), jnp.float32),
            pltpu.VMEM((rows_per_tile, c), jnp.float32),
            pltpu.VMEM((_S,), jnp.int32),
            pltpu.VMEM((_S,), jnp.int32),
            pltpu.VMEM((_S, c), jnp.float32),
            pltpu.VMEM((448,), jnp.int32),
            pltpu.VMEM((448,), jnp.int32),
            pltpu.SemaphoreType.DMA,
        ),
        compiler_params=pltpu.CompilerParams(use_tc_tiling_on_sc=False),
    )
    def scat_k(msgs, oflat, dense, bnd, ps, out, spm, stage, oid_v, lid_v,
               rows_v, bnd_v, ps_v, lsem):
        cid = lax.axis_index("c")
        sid = lax.axis_index("s")
        pltpu.sync_copy(bnd, bnd_v)
        pltpu.sync_copy(ps, ps_v)
        for ch in range(nch):
            @pl.when(cid == (ch % 2))
            def _chunk():
                r0 = ch * _CH + sid * rows_per_tile
                l0 = sid * rows_per_tile
                pltpu.sync_copy(dense.at[pl.ds(r0, rows_per_tile)], stage)
                pltpu.sync_copy(stage, spm.at[pl.ds(l0, rows_per_tile)])
                plsc.subcore_barrier()

                tvec = ps_v[pl.ds(26 * 16, 16)]
                total = tvec[ch]
                t0 = total * sid // 16
                t1 = total * (sid + 1) // 16

                def k_body(k, carry):
                    bv = bnd_v[pl.ds(k * 16, 16)]
                    pk = ps_v[pl.ds(k * 16, 16)]
                    pk1 = ps_v[pl.ds(k * 16 + 16, 16)]
                    lo = bv[ch]
                    p0 = pk[ch]
                    p1 = pk1[ch]
                    o_lo = jnp.maximum(p0, t0)
                    o_hi = jnp.minimum(p1, t1)

                    @pl.when(o_lo < o_hi)
                    def _seg():
                        s_lo = lo + (o_lo - p0)
                        s_hi = lo + (o_hi - p0)
                        s0 = (s_lo // 8) * 8
                        nit = (s_hi - s0 + _S - 1) // _S

                        def it(i, c2):
                            s = s0 + i * _S
                            row0 = k * pp + s
                            d1 = pltpu.async_copy(
                                oflat.at[pl.ds(row0, _S)], oid_v, lsem)
                            d2 = pltpu.async_copy(
                                msgs.at[pl.ds(row0, _S)], rows_v, lsem)
                            d1.wait()
                            d2.wait()
                            for v in range(_S // 16):
                                vec = oid_v[pl.ds(v * 16, 16)]
                                pos = s + v * 16 + lax.broadcasted_iota(
                                    jnp.int32, (16,), 0)
                                ok = (pos >= s_lo) & (pos < s_hi)
                                lid_v[pl.ds(v * 16, 16)] = jnp.where(
                                    ok, vec - ch * _CH, _CH + (pos & 7))
                            pltpu.sync_copy(rows_v, spm.at[lid_v], add=True)
                            return c2

                        lax.fori_loop(0, nit, it, 0, unroll=False)
                    return carry

                lax.fori_loop(0, 26, k_body, 0, unroll=False)
                plsc.subcore_barrier()
                pltpu.sync_copy(spm.at[pl.ds(l0, rows_per_tile)], stage)
                pltpu.sync_copy(stage, out.at[pl.ds(r0, rows_per_tile)])

    return scat_k


def _msg_gemm(g2, w2, aff):
    tot2, _ = g2.shape
    noff = w2.shape[0]
    blk = 1024
    jblk = tot2 // noff // blk

    def body(*refs):
        if aff is None:
            g_ref, w_ref, o_ref = refs
            g = g_ref[...]
        else:
            g_ref, w_ref, a_ref, o_ref = refs
            g = g_ref[...]
            g = jnp.maximum(g * a_ref[0:1, :] + a_ref[1:2, :], 0.0)
        o_ref[...] = jnp.dot(g, w_ref[0], preferred_element_type=jnp.float32)

    in_specs = [
        pl.BlockSpec((blk, 128), lambda k, j: (k * jblk + j, 0)),
        pl.BlockSpec((1, 128, 128), lambda k, j: (k, 0, 0)),
    ]
    args = [g2, w2]
    if aff is not None:
        in_specs.append(pl.BlockSpec((2, 128), lambda k, j: (0, 0)))
        args.append(aff)
    return pl.pallas_call(
        body,
        grid=(noff, jblk),
        in_specs=in_specs,
        out_specs=pl.BlockSpec((blk, 128), lambda k, j: (k * jblk + j, 0)),
        out_shape=jax.ShapeDtypeStruct(g2.shape, jnp.float32),
    )(*args)


def _dense_gemm(x2, w13, b128, aff, out_rows):
    blk = 2048
    grid = (x2.shape[0] + blk - 1) // blk

    def body(*refs):
        if aff is None:
            x_ref, w_ref, b_ref, o_ref = refs
            v = x_ref[...]
        else:
            x_ref, w_ref, b_ref, a_ref, o_ref = refs
            v = x_ref[...]
            v = jnp.maximum(v * a_ref[0:1, :] + a_ref[1:2, :], 0.0)
        o_ref[...] = (
            jnp.dot(v, w_ref[...], preferred_element_type=jnp.float32)
            + b_ref[0:1, :]
        )

    in_specs = [
        pl.BlockSpec((blk, 128), lambda i: (i, 0)),
        pl.BlockSpec((128, 128), lambda i: (0, 0)),
        pl.BlockSpec((1, 128), lambda i: (0, 0)),
    ]
    args = [x2, w13, b128]
    if aff is not None:
        in_specs.append(pl.BlockSpec((2, 128), lambda i: (0, 0)))
        args.append(aff)
    return pl.pallas_call(
        body,
        grid=(grid,),
        in_specs=in_specs,
        out_specs=pl.BlockSpec((blk, 128), lambda i: (i, 0)),
        out_shape=jax.ShapeDtypeStruct((out_rows, 128), jnp.float32),
    )(*args)


def _bn_affine(hv, gvec, bevec, npair, nrows):
    blk = 2048
    grid = hv.shape[0] // blk

    def body(h_ref, g_ref, be_ref, o_ref):
        i = pl.program_id(0)

        @pl.when(i == 0)
        def _init():
            o_ref[...] = jnp.zeros_like(o_ref)

        h = h_ref[...]
        r = i * blk + lax.broadcasted_iota(jnp.int32, (blk, 1), 0)
        h = jnp.where(r < npair, h, 0.0)
        acc = jnp.concatenate(
            [jnp.sum(h, 0, keepdims=True), jnp.sum(h * h, 0, keepdims=True)], 0
        )
        o_ref[...] += acc

        @pl.when(i == grid - 1)
        def _fin():
            s = o_ref[0:1, :]
            q = o_ref[1:2, :]
            s64 = s[:, :64] + s[:, 64:]
            q64 = q[:, :64] + q[:, 64:]
            m = s64 / nrows
            var = q64 / nrows - m * m
            inv = lax.rsqrt(var + 1e-5)
            sc = inv * g_ref[...]
            sh = be_ref[...] - m * sc
            o_ref[...] = jnp.concatenate(
                [jnp.concatenate([sc, sc], 1), jnp.concatenate([sh, sh], 1)], 0
            )

    return pl.pallas_call(
        body,
        grid=(grid,),
        in_specs=[
            pl.BlockSpec((blk, 128), lambda i: (i, 0)),
            pl.BlockSpec((1, 64), lambda i: (0, 0)),
            pl.BlockSpec((1, 64), lambda i: (0, 0)),
        ],
        out_specs=pl.BlockSpec((2, 128), lambda i: (0, 0)),
        out_shape=jax.ShapeDtypeStruct((2, 128), jnp.float32),
    )(hv, gvec.reshape(1, 64), bevec.reshape(1, 64))


def _final(h2v, x2, aff2, npair):
    blk = 2048
    grid = (npair + blk - 1) // blk

    def body(h_ref, x_ref, a_ref, o_ref):
        h = h_ref[...]
        o_ref[...] = jnp.maximum(
            h * a_ref[0:1, :] + a_ref[1:2, :] + x_ref[...], 0.0
        )

    return pl.pallas_call(
        body,
        grid=(grid,),
        in_specs=[
            pl.BlockSpec((blk, 128), lambda i: (i, 0)),
            pl.BlockSpec((blk, 128), lambda i: (i, 0)),
            pl.BlockSpec((2, 128), lambda i: (0, 0)),
        ],
        out_specs=pl.BlockSpec((blk, 128), lambda i: (i, 0)),
        out_shape=jax.ShapeDtypeStruct((npair, 128), jnp.float32),
    )(h2v, x2, aff2)


def kernel(x, W1, b1, g1, be1, W2, b2, g2, be2, in_idx, out_idx):
    n, c = x.shape
    p = in_idx.shape[1]
    pp = -(-(p + _S) // 1024) * 1024
    tot = 26 * pp
    nch = -(-(n + 1) // _CH)
    hp = nch * _CH
    npair = n // 2

    in_pad = jnp.full((26, pp), n, jnp.int32)
    in_pad = in_pad.at[:, :p].set(in_idx)
    in_flat = in_pad.reshape(-1)
    spread = (jnp.arange(tot, dtype=jnp.int32) * 7919) % n
    in_flat = jnp.where(in_flat >= n, spread, in_flat)
    out_pad = jnp.full((26, pp), n, jnp.int32)
    out_pad = out_pad.at[:, :p].set(out_idx)
    out_flat = out_pad.reshape(-1)
    edges = jnp.array([min(i * _CH, n) for i in range(nch + 1)], jnp.int32)
    bounds2d = jnp.sum(
        out_pad[:, :, None] < edges[None, None, :], axis=1, dtype=jnp.int32
    )
    bnd_flat = (
        jnp.zeros((28, 16), jnp.int32).at[:26, : nch + 1].set(bounds2d).reshape(-1)
    )
    lens = bounds2d[:, 1:] - bounds2d[:, :-1]
    cs = jnp.concatenate(
        [jnp.zeros((1, nch), jnp.int32), jnp.cumsum(lens, 0, dtype=jnp.int32)], 0
    )
    ps_flat = (
        jnp.zeros((28, 16), jnp.int32).at[:27, :nch].set(cs).reshape(-1)
    )

    sel = [k for k in range(27) if k != 13]

    def blockdiag(w):
        z = jnp.zeros(w.shape[:-2] + (64, 64), w.dtype)
        top = jnp.concatenate([w, z], -1)
        bot = jnp.concatenate([z, w], -1)
        return jnp.concatenate([top, bot], -2)

    w1n = blockdiag(W1[jnp.array(sel)])
    w2n = blockdiag(W2[jnp.array(sel)])
    w1c = blockdiag(W1[13])
    w2c = blockdiag(W2[13])
    b1d = jnp.tile(b1, 2).reshape(1, 128)
    b2d = jnp.tile(b2, 2).reshape(1, 128)

    x2 = x.reshape(npair, 128)

    gather_x = _make_gather(n, tot, c)
    gather_h = _make_gather(hp, tot, c)
    scatter = _make_scatter(tot, pp, c, nch, n)

    g_rows = gather_x(x, in_flat)
    m1 = _msg_gemm(g_rows.reshape(tot // 2, 128), w1n, None)
    d1 = _dense_gemm(x2, w1c, b1d, None, hp // 2)
    h1 = scatter(m1.reshape(tot, c), out_flat, d1.reshape(hp, c), bnd_flat,
                 ps_flat)
    aff1 = _bn_affine(h1.reshape(hp // 2, 128), g1, be1, npair, n)

    g_rows2 = gather_h(h1, in_flat)
    m2 = _msg_gemm(g_rows2.reshape(tot // 2, 128), w2n, aff1)
    d2 = _dense_gemm(h1.reshape(hp // 2, 128), w2c, b2d, aff1, hp // 2)
    h2 = scatter(m2.reshape(tot, c), out_flat, d2.reshape(hp, c), bnd_flat,
                 ps_flat)
    aff2 = _bn_affine(h2.reshape(hp // 2, 128), g2, be2, npair, n)

    out = _final(h2.reshape(hp // 2, 128), x2, aff2, npair)
    return out.reshape(n, c)


# speedup vs baseline: 5.4839x; 1.0006x over previous
import functools

import jax
import jax.numpy as jnp
from jax import lax
from jax.experimental import pallas as pl
from jax.experimental.pallas import tpu as pltpu
from jax.experimental.pallas import tpu_sc as plsc

_S = 128
_CH = 8192


def _make_gather(n_src, tot, c, s=_S, gpb=5):
    _S = s
    nw = 32
    per_w = tot // nw
    gw = gpb * _S
    ngrp = per_w // gw
    assert per_w % gw == 0
    mesh = plsc.VectorSubcoreMesh(core_axis_name="c", subcore_axis_name="s")

    @functools.partial(
        pl.kernel,
        out_type=jax.ShapeDtypeStruct((tot, c), jnp.float32),
        mesh=mesh,
        scratch_types=(
            pltpu.VMEM((per_w,), jnp.int32),
            pltpu.VMEM((gw, c), jnp.float32),
            pltpu.VMEM((gw, c), jnp.float32),
            pltpu.SemaphoreType.DMA,
            pltpu.SemaphoreType.DMA,
        ),
        compiler_params=pltpu.CompilerParams(use_tc_tiling_on_sc=False),
    )
    def gather_k(src, idxf, out, idx_v, buf0, buf1, gsem, wsem):
        cid = lax.axis_index("c")
        sid = lax.axis_index("s")
        wid = sid * 2 + cid
        base = pl.multiple_of(wid * per_w, _S)
        pltpu.sync_copy(idxf.at[pl.ds(base, per_w)], idx_v)

        def issue(g, buf):
            return [
                pltpu.async_copy(
                    src.at[idx_v.at[pl.ds(g * gw + b * _S, _S)]],
                    buf.at[pl.ds(b * _S, _S)],
                    gsem,
                )
                for b in range(gpb)
            ]

        descs = issue(0, buf0)
        wprev = None
        for g in range(ngrp):
            cur, nxt = (buf0, buf1) if g % 2 == 0 else (buf1, buf0)
            for d in descs:
                d.wait()
            if g + 1 < ngrp:
                descs = issue(g + 1, nxt)
            if wprev is not None:
                wprev.wait()
            wprev = pltpu.async_copy(
                cur, out.at[pl.ds(base + g * gw, gw)], wsem
            )
        wprev.wait()

    return gather_k


def _make_scatter(tot, pp, c, nch, n):
    mesh = plsc.VectorSubcoreMesh(core_axis_name="c", subcore_axis_name="s")
    rows_per_tile = _CH // 16

    @functools.partial(
        pl.kernel,
        out_type=jax.ShapeDtypeStruct((nch * _CH, c), jnp.float32),
        mesh=mesh,
        scratch_types=(
            pltpu.VMEM_SHARED((_CH + 8, c), jnp.float32),
            pltpu.VMEM((rows_per_tile, c), jnp.float32),
            pltpu.VMEM((_S,), jnp.int32),
            pltpu.VMEM((_S,), jnp.int32),
            pltpu.VMEM((_S, c), jnp.float32),
            pltpu.VMEM((448,), jnp.int32),
            pltpu.VMEM((448,), jnp.int32),
            pltpu.SemaphoreType.DMA,
        ),
        compiler_params=pltpu.CompilerParams(use_tc_tiling_on_sc=False),
    )
    def scat_k(msgs, oflat, dense, bnd, ps, out, spm, stage, oid_v, lid_v,
               rows_v, bnd_v, ps_v, lsem):
        cid = lax.axis_index("c")
        sid = lax.axis_index("s")
        pltpu.sync_copy(bnd, bnd_v)
        pltpu.sync_copy(ps, ps_v)
        for ch in range(nch):
            @pl.when(cid == (ch % 2))
            def _chunk():
                r0 = ch * _CH + sid * rows_per_tile
                l0 = sid * rows_per_tile
                pltpu.sync_copy(dense.at[pl.ds(r0, rows_per_tile)], stage)
                pltpu.sync_copy(stage, spm.at[pl.ds(l0, rows_per_tile)])
                plsc.subcore_barrier()

                tvec = ps_v[pl.ds(26 * 16, 16)]
                total = tvec[ch]
                t0 = total * sid // 16
                t1 = total * (sid + 1) // 16

                def k_body(k, carry):
                    bv = bnd_v[pl.ds(k * 16, 16)]
                    pk = ps_v[pl.ds(k * 16, 16)]
                    pk1 = ps_v[pl.ds(k * 16 + 16, 16)]
                    lo = bv[ch]
                    p0 = pk[ch]
                    p1 = pk1[ch]
                    o_lo = jnp.maximum(p0, t0)
                    o_hi = jnp.minimum(p1, t1)

                    @pl.when(o_lo < o_hi)
                    def _seg():
                        s_lo = lo + (o_lo - p0)
                        s_hi = lo + (o_hi - p0)
                        s0 = (s_lo // 8) * 8
                        nit = (s_hi - s0 + _S - 1) // _S

                        def it(i, c2):
                            s = s0 + i * _S
                            row0 = k * pp + s
                            d1 = pltpu.async_copy(
                                oflat.at[pl.ds(row0, _S)], oid_v, lsem)
                            d2 = pltpu.async_copy(
                                msgs.at[pl.ds(row0, _S)], rows_v, lsem)
                            d1.wait()
                            d2.wait()
                            for v in range(_S // 16):
                                vec = oid_v[pl.ds(v * 16, 16)]
                                pos = s + v * 16 + lax.broadcasted_iota(
                                    jnp.int32, (16,), 0)
                                ok = (pos >= s_lo) & (pos < s_hi)
                                lid_v[pl.ds(v * 16, 16)] = jnp.where(
                                    ok, vec - ch * _CH, _CH + (pos & 7))
                            pltpu.sync_copy(rows_v, spm.at[lid_v], add=True)
                            return c2

                        lax.fori_loop(0, nit, it, 0, unroll=False)
                    return carry

                lax.fori_loop(0, 26, k_body, 0, unroll=False)
                plsc.subcore_barrier()
                pltpu.sync_copy(spm.at[pl.ds(l0, rows_per_tile)], stage)
                pltpu.sync_copy(stage, out.at[pl.ds(r0, rows_per_tile)])

    return scat_k


def _msg_gemm(g2, w2, aff):
    tot2, _ = g2.shape
    noff = w2.shape[0]
    blk = 1024
    jblk = tot2 // noff // blk

    def body(*refs):
        if aff is None:
            g_ref, w_ref, o_ref = refs
            g = g_ref[...]
        else:
            g_ref, w_ref, a_ref, o_ref = refs
            g = g_ref[...]
            g = jnp.maximum(g * a_ref[0:1, :] + a_ref[1:2, :], 0.0)
        o_ref[...] = jnp.dot(g, w_ref[0], preferred_element_type=jnp.float32)

    in_specs = [
        pl.BlockSpec((blk, 128), lambda k, j: (k * jblk + j, 0)),
        pl.BlockSpec((1, 128, 128), lambda k, j: (k, 0, 0)),
    ]
    args = [g2, w2]
    if aff is not None:
        in_specs.append(pl.BlockSpec((2, 128), lambda k, j: (0, 0)))
        args.append(aff)
    return pl.pallas_call(
        body,
        grid=(noff, jblk),
        in_specs=in_specs,
        out_specs=pl.BlockSpec((blk, 128), lambda k, j: (k * jblk + j, 0)),
        out_shape=jax.ShapeDtypeStruct(g2.shape, jnp.float32),
    )(*args)


def _dense_gemm(x2, w13, b128, aff, out_rows):
    blk = 2048
    grid = (x2.shape[0] + blk - 1) // blk

    def body(*refs):
        if aff is None:
            x_ref, w_ref, b_ref, o_ref = refs
            v = x_ref[...]
        else:
            x_ref, w_ref, b_ref, a_ref, o_ref = refs
            v = x_ref[...]
            v = jnp.maximum(v * a_ref[0:1, :] + a_ref[1:2, :], 0.0)
        o_ref[...] = (
            jnp.dot(v, w_ref[...], preferred_element_type=jnp.float32)
            + b_ref[0:1, :]
        )

    in_specs = [
        pl.BlockSpec((blk, 128), lambda i: (i, 0)),
        pl.BlockSpec((128, 128), lambda i: (0, 0)),
        pl.BlockSpec((1, 128), lambda i: (0, 0)),
    ]
    args = [x2, w13, b128]
    if aff is not None:
        in_specs.append(pl.BlockSpec((2, 128), lambda i: (0, 0)))
        args.append(aff)
    return pl.pallas_call(
        body,
        grid=(grid,),
        in_specs=in_specs,
        out_specs=pl.BlockSpec((blk, 128), lambda i: (i, 0)),
        out_shape=jax.ShapeDtypeStruct((out_rows, 128), jnp.float32),
    )(*args)


def _bn_affine(hv, gvec, bevec, npair, nrows):
    blk = 2048
    grid = hv.shape[0] // blk

    def body(h_ref, g_ref, be_ref, o_ref):
        i = pl.program_id(0)

        @pl.when(i == 0)
        def _init():
            o_ref[...] = jnp.zeros_like(o_ref)

        h = h_ref[...]
        r = i * blk + lax.broadcasted_iota(jnp.int32, (blk, 1), 0)
        h = jnp.where(r < npair, h, 0.0)
        acc = jnp.concatenate(
            [jnp.sum(h, 0, keepdims=True), jnp.sum(h * h, 0, keepdims=True)], 0
        )
        o_ref[...] += acc

        @pl.when(i == grid - 1)
        def _fin():
            s = o_ref[0:1, :]
            q = o_ref[1:2, :]
            s64 = s[:, :64] + s[:, 64:]
            q64 = q[:, :64] + q[:, 64:]
            m = s64 / nrows
            var = q64 / nrows - m * m
            inv = lax.rsqrt(var + 1e-5)
            sc = inv * g_ref[...]
            sh = be_ref[...] - m * sc
            o_ref[...] = jnp.concatenate(
                [jnp.concatenate([sc, sc], 1), jnp.concatenate([sh, sh], 1)], 0
            )

    return pl.pallas_call(
        body,
        grid=(grid,),
        in_specs=[
            pl.BlockSpec((blk, 128), lambda i: (i, 0)),
            pl.BlockSpec((1, 64), lambda i: (0, 0)),
            pl.BlockSpec((1, 64), lambda i: (0, 0)),
        ],
        out_specs=pl.BlockSpec((2, 128), lambda i: (0, 0)),
        out_shape=jax.ShapeDtypeStruct((2, 128), jnp.float32),
    )(hv, gvec.reshape(1, 64), bevec.reshape(1, 64))


def _final(h2v, x2, aff2, npair):
    blk = 2048
    grid = (npair + blk - 1) // blk

    def body(h_ref, x_ref, a_ref, o_ref):
        h = h_ref[...]
        o_ref[...] = jnp.maximum(
            h * a_ref[0:1, :] + a_ref[1:2, :] + x_ref[...], 0.0
        )

    return pl.pallas_call(
        body,
        grid=(grid,),
        in_specs=[
            pl.BlockSpec((blk, 128), lambda i: (i, 0)),
            pl.BlockSpec((blk, 128), lambda i: (i, 0)),
            pl.BlockSpec((2, 128), lambda i: (0, 0)),
        ],
        out_specs=pl.BlockSpec((blk, 128), lambda i: (i, 0)),
        out_shape=jax.ShapeDtypeStruct((npair, 128), jnp.float32),
    )(h2v, x2, aff2)


def kernel(x, W1, b1, g1, be1, W2, b2, g2, be2, in_idx, out_idx):
    n, c = x.shape
    p = in_idx.shape[1]
    pp = -(-(p + _S) // 1024) * 1024
    tot = 26 * pp
    nch = -(-(n + 1) // _CH)
    hp = nch * _CH
    npair = n // 2

    in_pad = jnp.full((26, pp), n, jnp.int32)
    in_pad = in_pad.at[:, :p].set(in_idx)
    in_flat = in_pad.reshape(-1)
    spread = (jnp.arange(tot, dtype=jnp.int32) * 7919) % n
    in_flat = jnp.where(in_flat >= n, spread, in_flat)
    out_pad = jnp.full((26, pp), n, jnp.int32)
    out_pad = out_pad.at[:, :p].set(out_idx)
    out_flat = out_pad.reshape(-1)
    edges = jnp.array([min(i * _CH, n) for i in range(nch + 1)], jnp.int32)
    bounds2d = jnp.sum(
        out_pad[:, :, None] < edges[None, None, :], axis=1, dtype=jnp.int32
    )
    bnd_flat = (
        jnp.zeros((28, 16), jnp.int32).at[:26, : nch + 1].set(bounds2d).reshape(-1)
    )
    lens = bounds2d[:, 1:] - bounds2d[:, :-1]
    cs = jnp.concatenate(
        [jnp.zeros((1, nch), jnp.int32), jnp.cumsum(lens, 0, dtype=jnp.int32)], 0
    )
    ps_flat = (
        jnp.zeros((28, 16), jnp.int32).at[:27, :nch].set(cs).reshape(-1)
    )

    sel = jnp.array([k for k in range(27) if k != 13])

    def blockdiag(w):
        z = jnp.zeros(w.shape[:-2] + (64, 64), w.dtype)
        top = jnp.concatenate([w, z], -1)
        bot = jnp.concatenate([z, w], -1)
        return jnp.concatenate([top, bot], -2)

    w1n = blockdiag(W1[sel])
    w2n = blockdiag(W2[sel])
    w1c = blockdiag(W1[13])
    w2c = blockdiag(W2[13])
    b1d = jnp.tile(b1, 2).reshape(1, 128)
    b2d = jnp.tile(b2, 2).reshape(1, 128)

    x2 = x.reshape(npair, 128)

    gather_x = _make_gather(n, tot, c)
    gather_h = _make_gather(hp, tot, c)
    scatter = _make_scatter(tot, pp, c, nch, n)

    g_rows = gather_x(x, in_flat)
    m1 = _msg_gemm(g_rows.reshape(tot // 2, 128), w1n, None)
    d1 = _dense_gemm(x2, w1c, b1d, None, hp // 2)
    h1 = scatter(m1.reshape(tot, c), out_flat, d1.reshape(hp, c), bnd_flat,
                 ps_flat)
    aff1 = _bn_affine(h1.reshape(hp // 2, 128), g1, be1, npair, n)

    g_rows2 = gather_h(h1, in_flat)
    m2 = _msg_gemm(g_rows2.reshape(tot // 2, 128), w2n, aff1)
    d2 = _dense_gemm(h1.reshape(hp // 2, 128), w2c, b2d, aff1, hp // 2)
    h2 = scatter(m2.reshape(tot, c), out_flat, d2.reshape(hp, c), bnd_flat,
                 ps_flat)
    aff2 = _bn_affine(h2.reshape(hp // 2, 128), g2, be2, npair, n)

    out = _final(h2.reshape(hp // 2, 128), x2, aff2, npair)
    return out.reshape(n, c)


# speedup vs baseline: 6.0015x; 1.0944x over previous
import functools

import jax
import jax.numpy as jnp
from jax import lax
from jax.experimental import pallas as pl
from jax.experimental.pallas import tpu as pltpu
from jax.experimental.pallas import tpu_sc as plsc

_S = 128
_CH = 8192


def _make_gather(n_src, tot, c, s=_S, gpb=5):
    _S = s
    nw = 32
    per_w = tot // nw
    gw = gpb * _S
    ngrp = per_w // gw
    assert per_w % gw == 0
    mesh = plsc.VectorSubcoreMesh(core_axis_name="c", subcore_axis_name="s")

    @functools.partial(
        pl.kernel,
        out_type=jax.ShapeDtypeStruct((tot, c), jnp.float32),
        mesh=mesh,
        scratch_types=(
            pltpu.VMEM((per_w,), jnp.int32),
            pltpu.VMEM((gw, c), jnp.float32),
            pltpu.VMEM((gw, c), jnp.float32),
            pltpu.SemaphoreType.DMA,
            pltpu.SemaphoreType.DMA,
        ),
        compiler_params=pltpu.CompilerParams(use_tc_tiling_on_sc=False),
    )
    def gather_k(src, idxf, out, idx_v, buf0, buf1, gsem, wsem):
        cid = lax.axis_index("c")
        sid = lax.axis_index("s")
        wid = sid * 2 + cid
        base = pl.multiple_of(wid * per_w, _S)
        pltpu.sync_copy(idxf.at[pl.ds(base, per_w)], idx_v)

        def issue(g, buf):
            return [
                pltpu.async_copy(
                    src.at[idx_v.at[pl.ds(g * gw + b * _S, _S)]],
                    buf.at[pl.ds(b * _S, _S)],
                    gsem,
                )
                for b in range(gpb)
            ]

        descs = issue(0, buf0)
        wprev = None
        for g in range(ngrp):
            cur, nxt = (buf0, buf1) if g % 2 == 0 else (buf1, buf0)
            for d in descs:
                d.wait()
            if g + 1 < ngrp:
                descs = issue(g + 1, nxt)
            if wprev is not None:
                wprev.wait()
            wprev = pltpu.async_copy(
                cur, out.at[pl.ds(base + g * gw, gw)], wsem
            )
        wprev.wait()

    return gather_k


def _make_scatter(tot, pp, c, nch, n):
    mesh = plsc.VectorSubcoreMesh(core_axis_name="c", subcore_axis_name="s")
    rows_per_tile = _CH // 16

    @functools.partial(
        pl.kernel,
        out_type=jax.ShapeDtypeStruct((nch * _CH, c), jnp.float32),
        mesh=mesh,
        scratch_types=(
            pltpu.VMEM_SHARED((_CH + 8, c), jnp.float32),
            pltpu.VMEM((rows_per_tile, c), jnp.float32),
            pltpu.VMEM((_S,), jnp.int32),
            pltpu.VMEM((_S,), jnp.int32),
            pltpu.VMEM((_S, c), jnp.float32),
            pltpu.VMEM((448,), jnp.int32),
            pltpu.VMEM((448,), jnp.int32),
            pltpu.SemaphoreType.DMA,
        ),
        compiler_params=pltpu.CompilerParams(use_tc_tiling_on_sc=False),
    )
    def scat_k(msgs, oflat, dense, bnd, ps, out, spm, stage, oid_v, lid_v,
               rows_v, bnd_v, ps_v, lsem):
        cid = lax.axis_index("c")
        sid = lax.axis_index("s")
        pltpu.sync_copy(bnd, bnd_v)
        pltpu.sync_copy(ps, ps_v)
        for ch in range(nch):
            @pl.when(cid == (ch % 2))
            def _chunk():
                r0 = ch * _CH + sid * rows_per_tile
                l0 = sid * rows_per_tile
                pltpu.sync_copy(dense.at[pl.ds(r0, rows_per_tile)], stage)
                pltpu.sync_copy(stage, spm.at[pl.ds(l0, rows_per_tile)])
                plsc.subcore_barrier()

                tvec = ps_v[pl.ds(26 * 16, 16)]
                total = tvec[ch]
                t0 = total * sid // 16
                t1 = total * (sid + 1) // 16

                def k_body(k, carry):
                    bv = bnd_v[pl.ds(k * 16, 16)]
                    pk = ps_v[pl.ds(k * 16, 16)]
                    pk1 = ps_v[pl.ds(k * 16 + 16, 16)]
                    lo = bv[ch]
                    p0 = pk[ch]
                    p1 = pk1[ch]
                    o_lo = jnp.maximum(p0, t0)
                    o_hi = jnp.minimum(p1, t1)

                    @pl.when(o_lo < o_hi)
                    def _seg():
                        s_lo = lo + (o_lo - p0)
                        s_hi = lo + (o_hi - p0)
                        s0 = (s_lo // 8) * 8
                        nit = (s_hi - s0 + _S - 1) // _S

                        def it(i, c2):
                            s = s0 + i * _S
                            row0 = k * pp + s
                            d1 = pltpu.async_copy(
                                oflat.at[pl.ds(row0, _S)], oid_v, lsem)
                            d2 = pltpu.async_copy(
                                msgs.at[pl.ds(row0, _S)], rows_v, lsem)
                            d1.wait()
                            d2.wait()
                            for v in range(_S // 16):
                                vec = oid_v[pl.ds(v * 16, 16)]
                                pos = s + v * 16 + lax.broadcasted_iota(
                                    jnp.int32, (16,), 0)
                                ok = (pos >= s_lo) & (pos < s_hi)
                                lid_v[pl.ds(v * 16, 16)] = jnp.where(
                                    ok, vec - ch * _CH, _CH + (pos & 7))
                            pltpu.sync_copy(rows_v, spm.at[lid_v], add=True)
                            return c2

                        lax.fori_loop(0, nit, it, 0, unroll=False)
                    return carry

                lax.fori_loop(0, 26, k_body, 0, unroll=False)
                plsc.subcore_barrier()
                pltpu.sync_copy(spm.at[pl.ds(l0, rows_per_tile)], stage)
                pltpu.sync_copy(stage, out.at[pl.ds(r0, rows_per_tile)])

    return scat_k


def _msg_gemm(g2, w2, aff):
    tot2, _ = g2.shape
    noff = w2.shape[0]
    blk = 2560
    jblk = tot2 // noff // blk

    def body(*refs):
        if aff is None:
            g_ref, w_ref, o_ref = refs
            g = g_ref[...]
        else:
            g_ref, w_ref, a_ref, o_ref = refs
            g = g_ref[...]
            g = jnp.maximum(g * a_ref[0:1, :] + a_ref[1:2, :], 0.0)
        o_ref[...] = jnp.dot(g, w_ref[0], preferred_element_type=jnp.float32)

    in_specs = [
        pl.BlockSpec((blk, 128), lambda k, j: (k * jblk + j, 0)),
        pl.BlockSpec((1, 128, 128), lambda k, j: (k, 0, 0)),
    ]
    args = [g2, w2]
    if aff is not None:
        in_specs.append(pl.BlockSpec((2, 128), lambda k, j: (0, 0)))
        args.append(aff)
    return pl.pallas_call(
        body,
        grid=(noff, jblk),
        in_specs=in_specs,
        out_specs=pl.BlockSpec((blk, 128), lambda k, j: (k * jblk + j, 0)),
        out_shape=jax.ShapeDtypeStruct(g2.shape, jnp.float32),
    )(*args)


def _dense_gemm(x2, w13, b128, aff, out_rows):
    blk = 2048
    grid = (x2.shape[0] + blk - 1) // blk

    def body(*refs):
        if aff is None:
            x_ref, w_ref, b_ref, o_ref = refs
            v = x_ref[...]
        else:
            x_ref, w_ref, b_ref, a_ref, o_ref = refs
            v = x_ref[...]
            v = jnp.maximum(v * a_ref[0:1, :] + a_ref[1:2, :], 0.0)
        o_ref[...] = (
            jnp.dot(v, w_ref[...], preferred_element_type=jnp.float32)
            + b_ref[0:1, :]
        )

    in_specs = [
        pl.BlockSpec((blk, 128), lambda i: (i, 0)),
        pl.BlockSpec((128, 128), lambda i: (0, 0)),
        pl.BlockSpec((1, 128), lambda i: (0, 0)),
    ]
    args = [x2, w13, b128]
    if aff is not None:
        in_specs.append(pl.BlockSpec((2, 128), lambda i: (0, 0)))
        args.append(aff)
    return pl.pallas_call(
        body,
        grid=(grid,),
        in_specs=in_specs,
        out_specs=pl.BlockSpec((blk, 128), lambda i: (i, 0)),
        out_shape=jax.ShapeDtypeStruct((out_rows, 128), jnp.float32),
    )(*args)


def _bn_affine(hv, gvec, bevec, npair, nrows):
    blk = 2048
    grid = hv.shape[0] // blk

    def body(h_ref, g_ref, be_ref, o_ref):
        i = pl.program_id(0)

        @pl.when(i == 0)
        def _init():
            o_ref[...] = jnp.zeros_like(o_ref)

        h = h_ref[...]
        r = i * blk + lax.broadcasted_iota(jnp.int32, (blk, 1), 0)
        h = jnp.where(r < npair, h, 0.0)
        acc = jnp.concatenate(
            [jnp.sum(h, 0, keepdims=True), jnp.sum(h * h, 0, keepdims=True)], 0
        )
        o_ref[...] += acc

        @pl.when(i == grid - 1)
        def _fin():
            s = o_ref[0:1, :]
            q = o_ref[1:2, :]
            s64 = s[:, :64] + s[:, 64:]
            q64 = q[:, :64] + q[:, 64:]
            m = s64 / nrows
            var = q64 / nrows - m * m
            inv = lax.rsqrt(var + 1e-5)
            sc = inv * g_ref[...]
            sh = be_ref[...] - m * sc
            o_ref[...] = jnp.concatenate(
                [jnp.concatenate([sc, sc], 1), jnp.concatenate([sh, sh], 1)], 0
            )

    return pl.pallas_call(
        body,
        grid=(grid,),
        in_specs=[
            pl.BlockSpec((blk, 128), lambda i: (i, 0)),
            pl.BlockSpec((1, 64), lambda i: (0, 0)),
            pl.BlockSpec((1, 64), lambda i: (0, 0)),
        ],
        out_specs=pl.BlockSpec((2, 128), lambda i: (0, 0)),
        out_shape=jax.ShapeDtypeStruct((2, 128), jnp.float32),
    )(hv, gvec.reshape(1, 64), bevec.reshape(1, 64))


def _final(h2v, x2, aff2, npair):
    blk = 2048
    grid = (npair + blk - 1) // blk

    def body(h_ref, x_ref, a_ref, o_ref):
        h = h_ref[...]
        o_ref[...] = jnp.maximum(
            h * a_ref[0:1, :] + a_ref[1:2, :] + x_ref[...], 0.0
        )

    return pl.pallas_call(
        body,
        grid=(grid,),
        in_specs=[
            pl.BlockSpec((blk, 128), lambda i: (i, 0)),
            pl.BlockSpec((blk, 128), lambda i: (i, 0)),
            pl.BlockSpec((2, 128), lambda i: (0, 0)),
        ],
        out_specs=pl.BlockSpec((blk, 128), lambda i: (i, 0)),
        out_shape=jax.ShapeDtypeStruct((npair, 128), jnp.float32),
    )(h2v, x2, aff2)


def kernel(x, W1, b1, g1, be1, W2, b2, g2, be2, in_idx, out_idx):
    n, c = x.shape
    p = in_idx.shape[1]
    pp = -(-(p + _S) // 1024) * 1024
    tot = 26 * pp
    nch = -(-(n + 1) // _CH)
    hp = nch * _CH
    npair = n // 2

    in_pad = jnp.full((26, pp), n, jnp.int32)
    in_pad = in_pad.at[:, :p].set(in_idx)
    in_flat = in_pad.reshape(-1)
    spread = (jnp.arange(tot, dtype=jnp.int32) * 7919) % n
    in_flat = jnp.where(in_flat >= n, spread, in_flat)
    out_pad = jnp.full((26, pp), n, jnp.int32)
    out_pad = out_pad.at[:, :p].set(out_idx)
    out_flat = out_pad.reshape(-1)
    edges = jnp.array([min(i * _CH, n) for i in range(nch + 1)], jnp.int32)
    bounds2d = jnp.sum(
        out_pad[:, :, None] < edges[None, None, :], axis=1, dtype=jnp.int32
    )
    bnd_flat = (
        jnp.zeros((28, 16), jnp.int32).at[:26, : nch + 1].set(bounds2d).reshape(-1)
    )
    lens = bounds2d[:, 1:] - bounds2d[:, :-1]
    cs = jnp.concatenate(
        [jnp.zeros((1, nch), jnp.int32), jnp.cumsum(lens, 0, dtype=jnp.int32)], 0
    )
    ps_flat = (
        jnp.zeros((28, 16), jnp.int32).at[:27, :nch].set(cs).reshape(-1)
    )

    sel = jnp.array([k for k in range(27) if k != 13])

    def blockdiag(w):
        z = jnp.zeros(w.shape[:-2] + (64, 64), w.dtype)
        top = jnp.concatenate([w, z], -1)
        bot = jnp.concatenate([z, w], -1)
        return jnp.concatenate([top, bot], -2)

    w1n = blockdiag(W1[sel])
    w2n = blockdiag(W2[sel])
    w1c = blockdiag(W1[13])
    w2c = blockdiag(W2[13])
    b1d = jnp.tile(b1, 2).reshape(1, 128)
    b2d = jnp.tile(b2, 2).reshape(1, 128)

    x2 = x.reshape(npair, 128)

    gather_x = _make_gather(n, tot, c)
    gather_h = _make_gather(hp, tot, c)
    scatter = _make_scatter(tot, pp, c, nch, n)

    g_rows = gather_x(x, in_flat)
    m1 = _msg_gemm(g_rows.reshape(tot // 2, 128), w1n, None)
    d1 = _dense_gemm(x2, w1c, b1d, None, hp // 2)
    h1 = scatter(m1.reshape(tot, c), out_flat, d1.reshape(hp, c), bnd_flat,
                 ps_flat)
    aff1 = _bn_affine(h1.reshape(hp // 2, 128), g1, be1, npair, n)

    g_rows2 = gather_h(h1, in_flat)
    m2 = _msg_gemm(g_rows2.reshape(tot // 2, 128), w2n, aff1)
    d2 = _dense_gemm(h1.reshape(hp // 2, 128), w2c, b2d, aff1, hp // 2)
    h2 = scatter(m2.reshape(tot, c), out_flat, d2.reshape(hp, c), bnd_flat,
                 ps_flat)
    aff2 = _bn_affine(h2.reshape(hp // 2, 128), g2, be2, npair, n)

    out = _final(h2.reshape(hp // 2, 128), x2, aff2, npair)
    return out.reshape(n, c)
